# Initial kernel scaffold; baseline (speedup 1.0000x reference)
import jax
import jax.numpy as jnp
from jax.experimental import pallas as pl


def kernel(node_features, mesh_senders, mesh_receivers, mesh_edge_features, world_senders, world_receivers, world_edge_features, W1_mesh, b1_mesh, W2_mesh, b2_mesh, g_mesh, beta_mesh, W1_world, b1_world, W2_world, b2_world, g_world, beta_world):
    raise NotImplementedError("write your pallas kernel here")


# speedup vs baseline: 4.5588x; 4.5588x over previous
import functools

import jax
import jax.numpy as jnp
from jax.experimental import pallas as pl
from jax.experimental.pallas import tpu as pltpu
from jax.experimental.pallas import tpu_sc as plsc

_GATHER_WINDOW = 128
_TC_BLOCK = 2000
_PRE_BLOCK = 2000


def _project_tables(node_features, w_stack):
    n, d = node_features.shape
    k = w_stack.shape[0]
    nb = n // _PRE_BLOCK

    def body(nf_ref, w_ref, out_ref):
        out_ref[...] = jnp.dot(nf_ref[...], w_ref[0],
                               preferred_element_type=jnp.float32)

    return pl.pallas_call(
        body,
        grid=(k, nb),
        in_specs=[
            pl.BlockSpec((_PRE_BLOCK, d), lambda j, i: (i, 0)),
            pl.BlockSpec((1, d, d), lambda j, i: (j, 0, 0)),
        ],
        out_specs=pl.BlockSpec((_PRE_BLOCK, d), lambda j, i: (j * nb + i, 0)),
        out_shape=jax.ShapeDtypeStruct((k * n, d), jnp.float32),
    )(node_features, w_stack)


def _sc_gather(table, idx):
    b = idx.shape[1]
    d = table.shape[1]
    mesh = plsc.VectorSubcoreMesh(core_axis_name="c", subcore_axis_name="s")

    @functools.partial(
        pl.kernel,
        out_type=jax.ShapeDtypeStruct((b, d), table.dtype),
        mesh=mesh,
    )
    def gather_kernel(t_hbm, i_hbm, o_hbm):
        def body(i_vmem, o_vmem):
            pltpu.sync_copy(t_hbm.at[i_vmem.at[0]], o_vmem)

        pltpu.emit_pipeline(
            body,
            grid=(b // _GATHER_WINDOW,),
            in_specs=[pl.BlockSpec((1, _GATHER_WINDOW), lambda i: (0, i))],
            out_specs=[pl.BlockSpec((_GATHER_WINDOW, d), lambda i: (i, 0))],
            core_axis_name=("c", "s"),
            dimension_semantics=(pltpu.PARALLEL,),
        )(i_hbm, o_hbm)

    return gather_kernel(table, idx)


def _edge_mlp(gathered, ef, w1c, b1, w2, b2, g, beta):
    e, d = ef.shape
    nb = e // _TC_BLOCK

    def body(sp_ref, rq_ref, ef_ref, w1c_ref, b1_ref, w2_ref, b2_ref,
             g_ref, beta_ref, out_ref):
        ef_blk = ef_ref[...]
        h = sp_ref[...] + rq_ref[...] + b1_ref[...]
        h = h + jnp.dot(ef_blk, w1c_ref[...],
                        preferred_element_type=jnp.float32)
        h = jnp.maximum(h, 0.0)
        y = jnp.dot(h, w2_ref[...], preferred_element_type=jnp.float32)
        y = y + b2_ref[...]
        mu = jnp.mean(y, axis=-1, keepdims=True)
        yc = y - mu
        var = jnp.mean(yc * yc, axis=-1, keepdims=True)
        out_ref[...] = (ef_blk
                        + yc * jax.lax.rsqrt(var + 1e-5) * g_ref[...]
                        + beta_ref[...])

    return pl.pallas_call(
        body,
        grid=(nb,),
        in_specs=[
            pl.BlockSpec((_TC_BLOCK, d), lambda i: (i, 0)),
            pl.BlockSpec((_TC_BLOCK, d), lambda i: (nb + i, 0)),
            pl.BlockSpec((_TC_BLOCK, d), lambda i: (i, 0)),
            pl.BlockSpec((d, d), lambda i: (0, 0)),
            pl.BlockSpec((1, d), lambda i: (0, 0)),
            pl.BlockSpec((d, d), lambda i: (0, 0)),
            pl.BlockSpec((1, d), lambda i: (0, 0)),
            pl.BlockSpec((1, d), lambda i: (0, 0)),
            pl.BlockSpec((1, d), lambda i: (0, 0)),
        ],
        out_specs=pl.BlockSpec((_TC_BLOCK, d), lambda i: (i, 0)),
        out_shape=jax.ShapeDtypeStruct((e, d), jnp.float32),
    )(gathered, gathered, ef, w1c, b1, w2, b2, g, beta)


def kernel(node_features, mesh_senders, mesh_receivers, mesh_edge_features,
           world_senders, world_receivers, world_edge_features,
           W1_mesh, b1_mesh, W2_mesh, b2_mesh, g_mesh, beta_mesh,
           W1_world, b1_world, W2_world, b2_world, g_world, beta_world):
    n, d = node_features.shape

    w_stack = jnp.stack([
        W1_mesh[:d], W1_mesh[d:2 * d],
        W1_world[:d], W1_world[d:2 * d],
    ])
    table = _project_tables(node_features, w_stack)

    idx_mesh = jnp.concatenate(
        [mesh_senders, mesh_receivers + n]).astype(jnp.int32).reshape(1, -1)
    idx_world = jnp.concatenate(
        [world_senders + 2 * n,
         world_receivers + 3 * n]).astype(jnp.int32).reshape(1, -1)

    rows_mesh = _sc_gather(table, idx_mesh)
    rows_world = _sc_gather(table, idx_world)

    mesh_out = _edge_mlp(
        rows_mesh, mesh_edge_features, W1_mesh[2 * d:],
        b1_mesh.reshape(1, d), W2_mesh, b2_mesh.reshape(1, d),
        g_mesh.reshape(1, d), beta_mesh.reshape(1, d))
    world_out = _edge_mlp(
        rows_world, world_edge_features, W1_world[2 * d:],
        b1_world.reshape(1, d), W2_world, b2_world.reshape(1, d),
        g_world.reshape(1, d), beta_world.reshape(1, d))
    return (mesh_out, world_out)


# speedup vs baseline: 4.5735x; 1.0032x over previous
import functools

import jax
import jax.numpy as jnp
from jax.experimental import pallas as pl
from jax.experimental.pallas import tpu as pltpu
from jax.experimental.pallas import tpu_sc as plsc

_GATHER_WINDOW = 128
_TC_BLOCK = 2000
_PRE_BLOCK = 2000


def _project_tables(node_features, w_stack):
    n, d = node_features.shape
    k = w_stack.shape[0]
    nb = n // _PRE_BLOCK

    def body(nf_ref, w_ref, out_ref):
        out_ref[...] = jnp.dot(nf_ref[...], w_ref[0],
                               preferred_element_type=jnp.float32)

    return pl.pallas_call(
        body,
        grid=(k, nb),
        in_specs=[
            pl.BlockSpec((_PRE_BLOCK, d), lambda j, i: (i, 0)),
            pl.BlockSpec((1, d, d), lambda j, i: (j, 0, 0)),
        ],
        out_specs=pl.BlockSpec((_PRE_BLOCK, d), lambda j, i: (j * nb + i, 0)),
        out_shape=jax.ShapeDtypeStruct((k * n, d), jnp.float32),
    )(node_features, w_stack)


def _sc_gather(table, idx):
    b = idx.shape[1]
    d = table.shape[1]
    mesh = plsc.VectorSubcoreMesh(core_axis_name="c", subcore_axis_name="s")

    @functools.partial(
        pl.kernel,
        out_type=jax.ShapeDtypeStruct((b, d), table.dtype),
        mesh=mesh,
    )
    def gather_kernel(t_hbm, i_hbm, o_hbm):
        def body(i_vmem, o_vmem):
            pltpu.sync_copy(t_hbm.at[i_vmem.at[0]], o_vmem)

        pltpu.emit_pipeline(
            body,
            grid=(b // _GATHER_WINDOW,),
            in_specs=[pl.BlockSpec((1, _GATHER_WINDOW), lambda i: (0, i))],
            out_specs=[pl.BlockSpec((_GATHER_WINDOW, d), lambda i: (i, 0))],
            core_axis_name=("c", "s"),
            dimension_semantics=(pltpu.PARALLEL,),
        )(i_hbm, o_hbm)

    return gather_kernel(table, idx)


def _edge_mlp(gathered, ef, w1c, b1, w2, b2, g, beta):
    e, d = ef.shape
    nb = e // _TC_BLOCK

    def body(sp_ref, rq_ref, ef_ref, w1c_ref, b1_ref, w2_ref, b2_ref,
             g_ref, beta_ref, out_ref):
        ef_blk = ef_ref[...]
        h = sp_ref[...] + rq_ref[...] + b1_ref[...]
        h = h + jnp.dot(ef_blk, w1c_ref[...],
                        preferred_element_type=jnp.float32)
        h = jnp.maximum(h, 0.0)
        y = jnp.dot(h, w2_ref[...], preferred_element_type=jnp.float32)
        y = y + b2_ref[...]
        mu = jnp.mean(y, axis=-1, keepdims=True)
        yc = y - mu
        var = jnp.mean(yc * yc, axis=-1, keepdims=True)
        out_ref[...] = (ef_blk
                        + yc * jax.lax.rsqrt(var + 1e-5) * g_ref[...]
                        + beta_ref[...])

    return pl.pallas_call(
        body,
        grid=(nb,),
        in_specs=[
            pl.BlockSpec((_TC_BLOCK, d), lambda i: (i, 0)),
            pl.BlockSpec((_TC_BLOCK, d), lambda i: (nb + i, 0)),
            pl.BlockSpec((_TC_BLOCK, d), lambda i: (i, 0)),
            pl.BlockSpec((d, d), lambda i: (0, 0)),
            pl.BlockSpec((1, d), lambda i: (0, 0)),
            pl.BlockSpec((d, d), lambda i: (0, 0)),
            pl.BlockSpec((1, d), lambda i: (0, 0)),
            pl.BlockSpec((1, d), lambda i: (0, 0)),
            pl.BlockSpec((1, d), lambda i: (0, 0)),
        ],
        out_specs=pl.BlockSpec((_TC_BLOCK, d), lambda i: (i, 0)),
        out_shape=jax.ShapeDtypeStruct((e, d), jnp.float32),
    )(gathered, gathered, ef, w1c, b1, w2, b2, g, beta)


def kernel(node_features, mesh_senders, mesh_receivers, mesh_edge_features,
           world_senders, world_receivers, world_edge_features,
           W1_mesh, b1_mesh, W2_mesh, b2_mesh, g_mesh, beta_mesh,
           W1_world, b1_world, W2_world, b2_world, g_world, beta_world):
    n, d = node_features.shape

    w_stack = jnp.stack([
        W1_mesh[:d], W1_mesh[d:2 * d],
        W1_world[:d], W1_world[d:2 * d],
    ])
    table = _project_tables(node_features, w_stack)

    idx_mesh = jnp.concatenate(
        [mesh_senders, mesh_receivers + n]).astype(jnp.int32).reshape(1, -1)
    idx_world = jnp.concatenate(
        [world_senders + 2 * n,
         world_receivers + 3 * n]).astype(jnp.int32).reshape(1, -1)

    rows_mesh = _sc_gather(table, idx_mesh)
    mesh_out = _edge_mlp(
        rows_mesh, mesh_edge_features, W1_mesh[2 * d:],
        b1_mesh.reshape(1, d), W2_mesh, b2_mesh.reshape(1, d),
        g_mesh.reshape(1, d), beta_mesh.reshape(1, d))
    rows_world = _sc_gather(table, idx_world)
    world_out = _edge_mlp(
        rows_world, world_edge_features, W1_world[2 * d:],
        b1_world.reshape(1, d), W2_world, b2_world.reshape(1, d),
        g_world.reshape(1, d), beta_world.reshape(1, d))
    return (mesh_out, world_out)
